# Initial kernel scaffold; baseline (speedup 1.0000x reference)
import jax
import jax.numpy as jnp
from jax.experimental import pallas as pl


def kernel(short_term_history, long_term_states, graph, W_lp, b_lp, W_sn, b_sn, W_ls, b_ls, g_ls, be_ls, W_g, b_g, W_tc, b_tc, W_out, b_out):
    raise NotImplementedError("write your pallas kernel here")


# speedup vs baseline: 59.7866x; 59.7866x over previous
import jax
import jax.numpy as jnp
from jax import lax
from jax.experimental import pallas as pl
from jax.experimental.pallas import tpu as pltpu
from jax.experimental.pallas import tpu_sc as plsc

B, L, N, C = 2, 12, 10000, 3
P, H = 8, 64
E = 160000
TRG, OUT = 12, 3

NP = 10240
NB = 1024
NW = 32
EPT = 5120
EPAD = NW * EPT
NCHUNK = EPT // 128
STRIPE = NP // 16
BH = B * H

f32 = jnp.float32


def _pre_body(long_ref, short_ref, w2b_ref, wbig_ref, beff_ref, g_ref, be_ref,
              wg_ref, x_ref):
  xs = []
  for b in range(B):
    ltp = jnp.dot(long_ref[b], w2b_ref[...], preferred_element_type=f32)
    yall = jnp.dot(short_ref[b], wbig_ref[...], preferred_element_type=f32)
    cols = []
    for l in range(L):
      y = yall[:, l * H:(l + 1) * H] + ltp + beff_ref[...]
      mu = jnp.mean(y, axis=1, keepdims=True)
      var = jnp.mean(jnp.square(y - mu), axis=1, keepdims=True)
      yn = (y - mu) * lax.rsqrt(var + 1e-5) * g_ref[...] + be_ref[...]
      cols.append(jnp.dot(yn, wg_ref[...], preferred_element_type=f32))
    xs.append(cols)
  for l in range(L):
    x_ref[l] = jnp.concatenate([xs[b][l] for b in range(B)], axis=1)


def _pre_stage(long2, short2, w2b, wbig, beff, g2, be2, wg):
  full = lambda shape: pl.BlockSpec(shape, lambda n: (0,) * len(shape))
  return pl.pallas_call(
      _pre_body,
      grid=(NP // NB,),
      in_specs=[
          pl.BlockSpec((B, NB, P * H), lambda n: (0, n, 0)),
          pl.BlockSpec((B, NB, L * C), lambda n: (0, n, 0)),
          full((P * H, H)),
          full((L * C, L * H)),
          full((1, H)),
          full((1, H)),
          full((1, H)),
          full((H, H)),
      ],
      out_specs=pl.BlockSpec((L, NB, BH), lambda n: (0, n, 0)),
      out_shape=jax.ShapeDtypeStruct((L, NP, BH), f32),
  )(long2, short2, w2b, wbig, beff, g2, be2, wg)


def _sc_body(x_hbm, graph_hbm, agg_out,
             sidx_all, didx_all, gidx0, gidx1, didx,
             rows0, rows1, acc, sem):
  c = lax.axis_index("c")
  t = lax.axis_index("s")
  w = c * 16 + t
  row0 = t * STRIPE

  def fill_rows(buf, val):
    def fill(r, _):
      for k in range(BH // 16):
        buf[r, pl.ds(k * 16, 16)] = jnp.full((16,), val, f32)
      return 0
    lax.fori_loop(0, 128, fill, 0)

  pltpu.sync_copy(graph_hbm.at[0, pl.ds(w * EPT, EPT)], sidx_all)
  pltpu.sync_copy(graph_hbm.at[1, pl.ds(w * EPT, EPT)], didx_all)

  def copy_gidx(buf, j, off):
    for k in range(8):
      buf[pl.ds(k * 16, 16)] = sidx_all[pl.ds(j * 128 + k * 16, 16)] + off

  def copy_didx(j):
    for k in range(8):
      didx[pl.ds(k * 16, 16)] = didx_all[pl.ds(j * 128 + k * 16, 16)]

  def per_l(l, _):
    fill_rows(rows0, 0.0)
    for i in range(STRIPE // 128):
      pltpu.sync_copy(rows0, acc.at[pl.ds(row0 + i * 128, 128)])
    plsc.subcore_barrier()

    @pl.when(l < L)
    def _():
      off = l * NP
      copy_gidx(gidx0, 0, off)
      pltpu.async_copy(x_hbm.at[gidx0], rows0, sem)

      def chunk_pair(gp, _):
        for phase in range(2):
          j = gp * 2 + phase
          cur_g, cur_r = (gidx0, rows0) if phase == 0 else (gidx1, rows1)
          nxt_g, nxt_r = (gidx1, rows1) if phase == 0 else (gidx0, rows0)

          @pl.when(j < NCHUNK - 1)
          def _():
            copy_gidx(nxt_g, j + 1, off)
            pltpu.async_copy(x_hbm.at[nxt_g], nxt_r, sem)

          pltpu.make_async_copy(x_hbm.at[cur_g], cur_r, sem).wait()
          copy_didx(j)
          pltpu.sync_copy(cur_r, acc.at[didx], add=True)
        return 0
      lax.fori_loop(0, NCHUNK // 2, chunk_pair, 0)

    @pl.when(l == L)
    def _():
      fill_rows(rows1, 1.0)

      def deg_chunk(j, _):
        copy_didx(j)
        pltpu.sync_copy(rows1, acc.at[didx], add=True)
        return 0
      lax.fori_loop(0, NCHUNK, deg_chunk, 0)

    plsc.subcore_barrier()
    pltpu.sync_copy(acc.at[pl.ds(row0, STRIPE)],
                    agg_out.at[pl.ds((c * (L + 1) + l) * NP + row0, STRIPE)])
    plsc.subcore_barrier()
    return 0

  lax.fori_loop(0, L + 1, per_l, 0)


def _sc_stage(x_flat, graph_p):
  mesh = plsc.VectorSubcoreMesh(core_axis_name="c", subcore_axis_name="s")
  call = pl.kernel(
      _sc_body,
      out_type=jax.ShapeDtypeStruct((2 * (L + 1) * NP, BH), f32),
      mesh=mesh,
      scratch_types=[
          pltpu.VMEM((EPT,), jnp.int32),
          pltpu.VMEM((EPT,), jnp.int32),
          pltpu.VMEM((128,), jnp.int32),
          pltpu.VMEM((128,), jnp.int32),
          pltpu.VMEM((128,), jnp.int32),
          pltpu.VMEM((128, BH), f32),
          pltpu.VMEM((128, BH), f32),
          pltpu.VMEM_SHARED((NP, BH), f32),
          pltpu.SemaphoreType.DMA,
      ],
  )
  return call(x_flat, graph_p)


def _post_body(agg_ref, wt_ref, bg_ref, btc_ref, wout_ref, bout_ref, o_ref):
  deg = agg_ref[0, L, :, :1] + agg_ref[1, L, :, :1]
  dinv = 1.0 / jnp.maximum(deg, 1.0)
  for b in range(B):
    sl = slice(b * H, (b + 1) * H)
    gs = []
    for l in range(L):
      a = agg_ref[0, l, :, sl] + agg_ref[1, l, :, sl]
      gs.append(jnp.maximum(a * dinv + bg_ref[...], 0.0))
    ts = []
    for l in range(L):
      s = None
      for k in range(3):
        src = l + k - 1
        if 0 <= src < L:
          term = jnp.dot(gs[src], wt_ref[k], preferred_element_type=f32)
          s = term if s is None else s + term
      ts.append(jnp.maximum(s + btc_ref[...], 0.0))
    cat = jnp.concatenate(ts, axis=1)
    o_ref[b] = jnp.tanh(
        jnp.dot(cat, wout_ref[...], preferred_element_type=f32) +
        bout_ref[...])


def _post_stage(agg4, wt, bg2, btc2, wout, bout2):
  full = lambda shape: pl.BlockSpec(shape, lambda n: (0,) * len(shape))
  return pl.pallas_call(
      _post_body,
      grid=(NP // NB,),
      in_specs=[
          pl.BlockSpec((2, L + 1, NB, BH), lambda n: (0, 0, n, 0)),
          full((3, H, H)),
          full((1, H)),
          full((1, H)),
          full((L * H, TRG * OUT)),
          full((1, TRG * OUT)),
      ],
      out_specs=pl.BlockSpec((B, NB, TRG * OUT), lambda n: (0, n, 0)),
      out_shape=jax.ShapeDtypeStruct((B, NP, TRG * OUT), f32),
  )(agg4, wt, bg2, btc2, wout, bout2)


def kernel(short_term_history, long_term_states, graph, W_lp, b_lp, W_sn,
           b_sn, W_ls, b_ls, g_ls, be_ls, W_g, b_g, W_tc, b_tc, W_out, b_out):
  wls_top, wls_bot = W_ls[:H], W_ls[H:]
  w2b = W_lp.transpose(2, 1, 0).reshape(P * H, H) @ wls_bot
  wbig = jnp.kron(jnp.eye(L, dtype=f32), W_sn @ wls_top)
  beff = (b_sn @ wls_top + b_lp @ wls_bot + b_ls)[None]
  wt = W_tc.transpose(2, 1, 0)

  pad = [(0, 0), (0, NP - N), (0, 0)]
  long2 = jnp.pad(long_term_states.reshape(B, N, P * H), pad)
  short2 = jnp.pad(
      short_term_history.transpose(0, 2, 1, 3).reshape(B, N, L * C), pad)

  x = _pre_stage(long2, short2, w2b, wbig, beff, g_ls[None], be_ls[None], W_g)
  x_flat = x.reshape(L * NP, BH)

  npad = EPAD - E
  pad_edges = jnp.stack([
      jnp.full((npad,), N, jnp.int32),
      jnp.full((npad,), NP - 1, jnp.int32),
  ])
  graph_p = jnp.concatenate([graph, pad_edges], axis=1)

  agg_flat = _sc_stage(x_flat, graph_p)

  out2 = _post_stage(agg_flat.reshape(2, L + 1, NP, BH), wt, b_g[None],
                     b_tc[None], W_out, b_out[None])
  pred = out2[:, :N].reshape(B, N, TRG, OUT)
  return pred.swapaxes(1, 2)


# speedup vs baseline: 59.8090x; 1.0004x over previous
import jax
import jax.numpy as jnp
from jax import lax
from jax.experimental import pallas as pl
from jax.experimental.pallas import tpu as pltpu
from jax.experimental.pallas import tpu_sc as plsc

B, L, N, C = 2, 12, 10000, 3
P, H = 8, 64
E = 160000
TRG, OUT = 12, 3

NP = 10240
NB = 1024
NW = 32
EPT = 5120
EPAD = NW * EPT
NCHUNK = EPT // 128
STRIPE = NP // 16
BH = B * H

f32 = jnp.float32


def _pre_body(long_ref, short_ref, w2b_ref, wbig_ref, beff_ref, g_ref, be_ref,
              wg_ref, x_ref):
  xs = []
  for b in range(B):
    ltp = jnp.dot(long_ref[b], w2b_ref[...], preferred_element_type=f32)
    yall = jnp.dot(short_ref[b], wbig_ref[...], preferred_element_type=f32)
    cols = []
    for l in range(L):
      y = yall[:, l * H:(l + 1) * H] + ltp + beff_ref[...]
      mu = jnp.mean(y, axis=1, keepdims=True)
      var = jnp.mean(jnp.square(y - mu), axis=1, keepdims=True)
      yn = (y - mu) * lax.rsqrt(var + 1e-5) * g_ref[...] + be_ref[...]
      cols.append(jnp.dot(yn, wg_ref[...], preferred_element_type=f32))
    xs.append(cols)
  for l in range(L):
    x_ref[l] = jnp.concatenate([xs[b][l] for b in range(B)], axis=1)


def _pre_stage(long2, short2, w2b, wbig, beff, g2, be2, wg):
  full = lambda shape: pl.BlockSpec(shape, lambda n: (0,) * len(shape))
  return pl.pallas_call(
      _pre_body,
      grid=(NP // NB,),
      in_specs=[
          pl.BlockSpec((B, NB, P * H), lambda n: (0, n, 0)),
          pl.BlockSpec((B, NB, L * C), lambda n: (0, n, 0)),
          full((P * H, H)),
          full((L * C, L * H)),
          full((1, H)),
          full((1, H)),
          full((1, H)),
          full((H, H)),
      ],
      out_specs=pl.BlockSpec((L, NB, BH), lambda n: (0, n, 0)),
      out_shape=jax.ShapeDtypeStruct((L, NP, BH), f32),
  )(long2, short2, w2b, wbig, beff, g2, be2, wg)


def _sc_body(x_hbm, graph_hbm, agg_out,
             sidx_all, didx_all, gidx, didx,
             rows, acc, gsem, ssem):
  c = lax.axis_index("c")
  t = lax.axis_index("s")
  w = c * 16 + t
  row0 = t * STRIPE

  def fill_rows(buf, val):
    def fill(r, _):
      for k in range(BH // 16):
        buf[r, pl.ds(k * 16, 16)] = jnp.full((16,), val, f32)
      return 0
    lax.fori_loop(0, 128, fill, 0)

  pltpu.sync_copy(graph_hbm.at[0, pl.ds(w * EPT, EPT)], sidx_all)
  pltpu.sync_copy(graph_hbm.at[1, pl.ds(w * EPT, EPT)], didx_all)

  def copy_gidx(p, j, off):
    for k in range(8):
      gidx[p, pl.ds(k * 16, 16)] = sidx_all[pl.ds(j * 128 + k * 16, 16)] + off

  def copy_didx(p, j):
    for k in range(8):
      didx[p, pl.ds(k * 16, 16)] = didx_all[pl.ds(j * 128 + k * 16, 16)]

  def wait_scatter(p):
    pltpu.make_async_copy(rows.at[p], acc.at[didx.at[p]], ssem).wait()

  def per_l(l, _):
    fill_rows(rows.at[0], 0.0)
    for i in range(STRIPE // 128):
      pltpu.sync_copy(rows.at[0], acc.at[pl.ds(row0 + i * 128, 128)])
    plsc.subcore_barrier()

    @pl.when(l < L)
    def _():
      off = l * NP
      copy_gidx(0, 0, off)
      pltpu.async_copy(x_hbm.at[gidx.at[0]], rows.at[0], gsem)

      def chunk_pair(gp, _):
        for p in range(2):
          j = gp * 2 + p
          q = 1 - p

          @pl.when(j >= 1)
          def _():
            wait_scatter(q)

          @pl.when(j + 1 < NCHUNK)
          def _():
            copy_gidx(q, j + 1, off)
            pltpu.async_copy(x_hbm.at[gidx.at[q]], rows.at[q], gsem)

          pltpu.make_async_copy(x_hbm.at[gidx.at[p]], rows.at[p],
                                gsem).wait()
          copy_didx(p, j)
          pltpu.make_async_copy(rows.at[p], acc.at[didx.at[p]],
                                ssem).start(add=True)
        return 0
      lax.fori_loop(0, NCHUNK // 2, chunk_pair, 0)
      wait_scatter((NCHUNK - 1) % 2)

    @pl.when(l == L)
    def _():
      fill_rows(rows.at[0], 1.0)

      def deg_chunk(j, _):
        copy_didx(0, j)
        pltpu.sync_copy(rows.at[0], acc.at[didx.at[0]], add=True)
        return 0
      lax.fori_loop(0, NCHUNK, deg_chunk, 0)

    plsc.subcore_barrier()
    pltpu.sync_copy(acc.at[pl.ds(row0, STRIPE)],
                    agg_out.at[pl.ds((c * (L + 1) + l) * NP + row0, STRIPE)])
    plsc.subcore_barrier()
    return 0

  lax.fori_loop(0, L + 1, per_l, 0)


def _sc_stage(x_flat, graph_p):
  mesh = plsc.VectorSubcoreMesh(core_axis_name="c", subcore_axis_name="s")
  call = pl.kernel(
      _sc_body,
      out_type=jax.ShapeDtypeStruct((2 * (L + 1) * NP, BH), f32),
      mesh=mesh,
      scratch_types=[
          pltpu.VMEM((EPT,), jnp.int32),
          pltpu.VMEM((EPT,), jnp.int32),
          pltpu.VMEM((2, 128), jnp.int32),
          pltpu.VMEM((2, 128), jnp.int32),
          pltpu.VMEM((2, 128, BH), f32),
          pltpu.VMEM_SHARED((NP, BH), f32),
          pltpu.SemaphoreType.DMA,
          pltpu.SemaphoreType.DMA,
      ],
  )
  return call(x_flat, graph_p)


def _post_body(agg_ref, wt_ref, bg_ref, btc_ref, wout_ref, bout_ref, o_ref):
  deg = agg_ref[0, L, :, :1] + agg_ref[1, L, :, :1]
  dinv = 1.0 / jnp.maximum(deg, 1.0)
  for b in range(B):
    sl = slice(b * H, (b + 1) * H)
    gs = []
    for l in range(L):
      a = agg_ref[0, l, :, sl] + agg_ref[1, l, :, sl]
      gs.append(jnp.maximum(a * dinv + bg_ref[...], 0.0))
    ts = []
    for l in range(L):
      s = None
      for k in range(3):
        src = l + k - 1
        if 0 <= src < L:
          term = jnp.dot(gs[src], wt_ref[k], preferred_element_type=f32)
          s = term if s is None else s + term
      ts.append(jnp.maximum(s + btc_ref[...], 0.0))
    cat = jnp.concatenate(ts, axis=1)
    o_ref[b] = jnp.tanh(
        jnp.dot(cat, wout_ref[...], preferred_element_type=f32) +
        bout_ref[...])


def _post_stage(agg4, wt, bg2, btc2, wout, bout2):
  full = lambda shape: pl.BlockSpec(shape, lambda n: (0,) * len(shape))
  return pl.pallas_call(
      _post_body,
      grid=(NP // NB,),
      in_specs=[
          pl.BlockSpec((2, L + 1, NB, BH), lambda n: (0, 0, n, 0)),
          full((3, H, H)),
          full((1, H)),
          full((1, H)),
          full((L * H, TRG * OUT)),
          full((1, TRG * OUT)),
      ],
      out_specs=pl.BlockSpec((B, NB, TRG * OUT), lambda n: (0, n, 0)),
      out_shape=jax.ShapeDtypeStruct((B, NP, TRG * OUT), f32),
  )(agg4, wt, bg2, btc2, wout, bout2)


def kernel(short_term_history, long_term_states, graph, W_lp, b_lp, W_sn,
           b_sn, W_ls, b_ls, g_ls, be_ls, W_g, b_g, W_tc, b_tc, W_out, b_out):
  wls_top, wls_bot = W_ls[:H], W_ls[H:]
  w2b = W_lp.transpose(2, 1, 0).reshape(P * H, H) @ wls_bot
  wbig = jnp.kron(jnp.eye(L, dtype=f32), W_sn @ wls_top)
  beff = (b_sn @ wls_top + b_lp @ wls_bot + b_ls)[None]
  wt = W_tc.transpose(2, 1, 0)

  pad = [(0, 0), (0, NP - N), (0, 0)]
  long2 = jnp.pad(long_term_states.reshape(B, N, P * H), pad)
  short2 = jnp.pad(
      short_term_history.transpose(0, 2, 1, 3).reshape(B, N, L * C), pad)

  x = _pre_stage(long2, short2, w2b, wbig, beff, g_ls[None], be_ls[None], W_g)
  x_flat = x.reshape(L * NP, BH)

  npad = EPAD - E
  pad_edges = jnp.stack([
      jnp.full((npad,), N, jnp.int32),
      jnp.full((npad,), NP - 1, jnp.int32),
  ])
  graph_p = jnp.concatenate([graph, pad_edges], axis=1)

  agg_flat = _sc_stage(x_flat, graph_p)

  out2 = _post_stage(agg_flat.reshape(2, L + 1, NP, BH), wt, b_g[None],
                     b_tc[None], W_out, b_out[None])
  pred = out2[:, :N].reshape(B, N, TRG, OUT)
  return pred.swapaxes(1, 2)
